# Initial kernel scaffold; baseline (speedup 1.0000x reference)
import jax
import jax.numpy as jnp
from jax.experimental import pallas as pl


def kernel(x, edge_index, pos_W, pos_b, Wl1, bl1, Wr1, Wl2, bl2, Wr2, Wl3, bl3, Wr3):
    raise NotImplementedError("write your pallas kernel here")


# speedup vs baseline: 27.7735x; 27.7735x over previous
import functools

import jax
import jax.numpy as jnp
from jax import lax
from jax.experimental import pallas as pl
from jax.experimental.pallas import tpu as pltpu
from jax.experimental.pallas import tpu_sc as plsc

N = 100000
E = 3200000
F = 16
SUB = 128
JSUB = 8
CHUNK = SUB * JSUB
NCH = E // CHUNK
NTILES = 32
RPT = N // 16
ZROWS = 1250


def _agg_body(src_hbm, dst_hbm, table_hbm, out_hbm,
              idx_s, idx_d, rows0, rows1, zbuf, sem0, sem1, acc):
    c = lax.axis_index("c")
    s = lax.axis_index("s")
    wid = s * 2 + c

    def zfill(i, _):
        zbuf[i] = jnp.zeros((F,), jnp.float32)
        return _
    lax.fori_loop(0, ZROWS, zfill, None)
    base = s * RPT
    for b in range(RPT // ZROWS):
        pltpu.sync_copy(zbuf, acc.at[pl.ds(base + b * ZROWS, ZROWS)])
    plsc.subcore_barrier()

    lo = (wid * NCH) // NTILES
    hi = ((wid + 1) * NCH) // NTILES

    def chunk_body(chunk, _):
        pltpu.sync_copy(src_hbm.at[chunk], idx_s)
        pltpu.sync_copy(dst_hbm.at[chunk], idx_d)
        bufs = (rows0, rows1)
        sems = (sem0, sem1)
        descs = [None] * JSUB
        descs[0] = pltpu.async_copy(table_hbm.at[idx_s.at[0]], bufs[0], sems[0])
        for j in range(JSUB):
            if j + 1 < JSUB:
                descs[j + 1] = pltpu.async_copy(
                    table_hbm.at[idx_s.at[j + 1]], bufs[(j + 1) % 2],
                    sems[(j + 1) % 2])
            descs[j].wait()
            pltpu.sync_copy(bufs[j % 2], acc.at[idx_d.at[j]], add=True)
        return _
    lax.fori_loop(lo, hi, chunk_body, None)
    plsc.subcore_barrier()

    pltpu.sync_copy(acc.at[pl.ds(base, RPT)],
                    out_hbm.at[c, pl.ds(base, RPT)])


def _agg(src, dst, table):
    mesh = plsc.VectorSubcoreMesh(core_axis_name="c", subcore_axis_name="s")
    k = pl.kernel(
        _agg_body,
        out_type=jax.ShapeDtypeStruct((2, N, F), jnp.float32),
        mesh=mesh,
        compiler_params=pltpu.CompilerParams(use_tc_tiling_on_sc=False),
        scratch_types=[
            pltpu.VMEM((JSUB, SUB), jnp.int32),
            pltpu.VMEM((JSUB, SUB), jnp.int32),
            pltpu.VMEM((SUB, F), jnp.float32),
            pltpu.VMEM((SUB, F), jnp.float32),
            pltpu.VMEM((ZROWS, F), jnp.float32),
            pltpu.SemaphoreType.DMA,
            pltpu.SemaphoreType.DMA,
            pltpu.VMEM_SHARED((N, F), jnp.float32),
        ],
    )
    return k(src, dst, table)


BLK = 4000
GRID = N // BLK


def _prep_body(x_ref, w_ref, b_ref, o_ref):
    i = pl.program_id(0)
    rows = (jnp.float32(i * BLK)
            + lax.broadcasted_iota(jnp.int32, (BLK, 1), 0).astype(jnp.float32))
    vect = jnp.tanh(rows * w_ref[...] + b_ref[...])
    o_ref[...] = jnp.concatenate(
        [x_ref[...], vect,
         jnp.ones((BLK, 1), jnp.float32),
         jnp.zeros((BLK, F - 9), jnp.float32)], axis=1)


def _prep(x, pos_W, pos_b):
    return pl.pallas_call(
        _prep_body,
        grid=(GRID,),
        in_specs=[
            pl.BlockSpec((BLK, 3), lambda i: (i, 0)),
            pl.BlockSpec((1, 5), lambda i: (0, 0)),
            pl.BlockSpec((1, 5), lambda i: (0, 0)),
        ],
        out_specs=pl.BlockSpec((BLK, F), lambda i: (i, 0)),
        out_shape=jax.ShapeDtypeStruct((N, F), jnp.float32),
    )(x, pos_W.reshape(1, 5), pos_b.reshape(1, 5))


def _dense1_body(p0, p1, h0, wl, bl, wr, h1_o, rcn_o):
    s8 = p0[:, :8] + p1[:, :8]
    cnt = p0[:, 8:9] + p1[:, 8:9]
    rcn = 1.0 / jnp.maximum(cnt, 1.0)
    h1_o[...] = (jnp.dot(s8 * rcn, wl[...], preferred_element_type=jnp.float32)
                 + bl[...]
                 + jnp.dot(h0[:, :8], wr[...],
                           preferred_element_type=jnp.float32))
    rcn_o[...] = rcn


def _dense1(p0, p1, h0ext, Wl, bl, Wr):
    return pl.pallas_call(
        _dense1_body,
        grid=(GRID,),
        in_specs=[
            pl.BlockSpec((BLK, F), lambda i: (i, 0)),
            pl.BlockSpec((BLK, F), lambda i: (i, 0)),
            pl.BlockSpec((BLK, F), lambda i: (i, 0)),
            pl.BlockSpec((8, F), lambda i: (0, 0)),
            pl.BlockSpec((1, F), lambda i: (0, 0)),
            pl.BlockSpec((8, F), lambda i: (0, 0)),
        ],
        out_specs=[
            pl.BlockSpec((BLK, F), lambda i: (i, 0)),
            pl.BlockSpec((BLK, 1), lambda i: (i, 0)),
        ],
        out_shape=[
            jax.ShapeDtypeStruct((N, F), jnp.float32),
            jax.ShapeDtypeStruct((N, 1), jnp.float32),
        ],
    )(p0, p1, h0ext, Wl.T, bl.reshape(1, F), Wr.T)


def _dense2_body(p0, p1, rcn, h, wl, bl, wr, o_ref):
    agg = (p0[...] + p1[...]) * rcn[...]
    o_ref[...] = (jnp.dot(agg, wl[...], preferred_element_type=jnp.float32)
                  + bl[...]
                  + jnp.dot(h[...], wr[...],
                            preferred_element_type=jnp.float32))


def _dense2(p0, p1, rcn, h, Wl, bl, Wr):
    return pl.pallas_call(
        _dense2_body,
        grid=(GRID,),
        in_specs=[
            pl.BlockSpec((BLK, F), lambda i: (i, 0)),
            pl.BlockSpec((BLK, F), lambda i: (i, 0)),
            pl.BlockSpec((BLK, 1), lambda i: (i, 0)),
            pl.BlockSpec((BLK, F), lambda i: (i, 0)),
            pl.BlockSpec((F, F), lambda i: (0, 0)),
            pl.BlockSpec((1, F), lambda i: (0, 0)),
            pl.BlockSpec((F, F), lambda i: (0, 0)),
        ],
        out_specs=pl.BlockSpec((BLK, F), lambda i: (i, 0)),
        out_shape=jax.ShapeDtypeStruct((N, F), jnp.float32),
    )(p0, p1, rcn, h, Wl.T, bl.reshape(1, F), Wr.T)


OBLK = 128


def _dense3_body(p0, p1, rcn, h, wl, bl, wr, o_ref):
    agg = (p0[...] + p1[...]) * rcn[...]
    o_ref[...] = (jnp.dot(agg, wl[...], preferred_element_type=jnp.float32)
                  + bl[...]
                  + jnp.dot(h[...], wr[...],
                            preferred_element_type=jnp.float32))


def _dense3(p0, p1, rcn, h, Wl, bl, Wr):
    return pl.pallas_call(
        _dense3_body,
        out_shape=jax.ShapeDtypeStruct((OBLK, 3), jnp.float32),
    )(p0, p1, rcn, h, Wl.T, bl.reshape(1, 3), Wr.T)


def kernel(x, edge_index, pos_W, pos_b,
           Wl1, bl1, Wr1, Wl2, bl2, Wr2, Wl3, bl3, Wr3):
    src = edge_index[0].reshape(NCH, JSUB, SUB)
    dst = edge_index[1].reshape(NCH, JSUB, SUB)

    h0ext = _prep(x, pos_W, pos_b)
    p = _agg(src, dst, h0ext)
    h1, rcn = _dense1(p[0], p[1], h0ext, Wl1, bl1, Wr1)
    p2 = _agg(src, dst, h1)
    h2 = _dense2(p2[0], p2[1], rcn, h1, Wl2, bl2, Wr2)
    p3 = _agg(src, dst, h2)
    out = _dense3(p3[0, :OBLK], p3[1, :OBLK], rcn[:OBLK], h2[:OBLK],
                  Wl3, bl3, Wr3)
    return out[:68]


# speedup vs baseline: 33.2290x; 1.1964x over previous
import functools

import jax
import jax.numpy as jnp
from jax import lax
from jax.experimental import pallas as pl
from jax.experimental.pallas import tpu as pltpu
from jax.experimental.pallas import tpu_sc as plsc

N = 100000
E = 3200000
F = 16
SUB = 128
JSUB = 8
CHUNK = SUB * JSUB
NCH = E // CHUNK
NTILES = 32
RPT = N // 16
ZROWS = 1250


def _agg_body(src_hbm, dst_hbm, table_hbm, out_hbm,
              idx_s, idx_d, rows0, rows1, zbuf, sem0, sem1, acc):
    c = lax.axis_index("c")
    s = lax.axis_index("s")
    wid = s * 2 + c

    def zfill(i, _):
        zbuf[i] = jnp.zeros((F,), jnp.float32)
        return _
    lax.fori_loop(0, ZROWS, zfill, None)
    base = s * RPT
    for b in range(RPT // ZROWS):
        pltpu.sync_copy(zbuf, acc.at[pl.ds(base + b * ZROWS, ZROWS)])
    plsc.subcore_barrier()

    lo = (wid * NCH) // NTILES
    hi = ((wid + 1) * NCH) // NTILES

    def chunk_body(chunk, _):
        pltpu.sync_copy(src_hbm.at[chunk], idx_s)
        pltpu.sync_copy(dst_hbm.at[chunk], idx_d)
        bufs = (rows0, rows1)
        sems = (sem0, sem1)
        descs = [None] * JSUB
        descs[0] = pltpu.async_copy(table_hbm.at[idx_s.at[0]], bufs[0], sems[0])
        for j in range(JSUB):
            if j + 1 < JSUB:
                descs[j + 1] = pltpu.async_copy(
                    table_hbm.at[idx_s.at[j + 1]], bufs[(j + 1) % 2],
                    sems[(j + 1) % 2])
            descs[j].wait()
            pltpu.sync_copy(bufs[j % 2], acc.at[idx_d.at[j]], add=True)
        return _
    lax.fori_loop(lo, hi, chunk_body, None)
    plsc.subcore_barrier()

    pltpu.sync_copy(acc.at[pl.ds(base, RPT)],
                    out_hbm.at[c, pl.ds(base, RPT)])


def _agg(src, dst, table):
    mesh = plsc.VectorSubcoreMesh(core_axis_name="c", subcore_axis_name="s")
    k = pl.kernel(
        _agg_body,
        out_type=jax.ShapeDtypeStruct((2, N, F), jnp.float32),
        mesh=mesh,
        compiler_params=pltpu.CompilerParams(use_tc_tiling_on_sc=False),
        scratch_types=[
            pltpu.VMEM((JSUB, SUB), jnp.int32),
            pltpu.VMEM((JSUB, SUB), jnp.int32),
            pltpu.VMEM((SUB, F), jnp.float32),
            pltpu.VMEM((SUB, F), jnp.float32),
            pltpu.VMEM((ZROWS, F), jnp.float32),
            pltpu.SemaphoreType.DMA,
            pltpu.SemaphoreType.DMA,
            pltpu.VMEM_SHARED((N, F), jnp.float32),
        ],
    )
    return k(src, dst, table)


NOUT = 68
OPAD = 80


def _agg68_body(src_hbm, dst_hbm, table_hbm, out_hbm, srcb, dstb, rowbuf, acc):
    c = lax.axis_index("c")
    s = lax.axis_index("s")
    wid = s * 2 + c

    def zfill(i, _):
        acc[i] = jnp.zeros((F,), jnp.float32)
        return _
    lax.fori_loop(0, OPAD, zfill, None)

    lo = (wid * NCH) // NTILES
    hi = ((wid + 1) * NCH) // NTILES

    def chunk_body(chunk, _):
        pltpu.sync_copy(dst_hbm.at[pl.ds(chunk * CHUNK, CHUNK)], dstb)

        def sub_body(j, __):
            jb = j * SUB
            dvs = [dstb[pl.ds(jb + k * 16, 16)] for k in range(8)]
            mins = functools.reduce(jnp.minimum, dvs)

            @pl.when(plsc.all_reduce_population_count(mins < NOUT)[0] > 0)
            def _hit():
                pltpu.sync_copy(src_hbm.at[pl.ds(chunk * CHUNK, CHUNK)], srcb)
                for k in range(8):
                    @pl.when(plsc.all_reduce_population_count(
                        dvs[k] < NOUT)[0] > 0)
                    def _grp(k=k):
                        sv = srcb[pl.ds(jb + k * 16, 16)]
                        for l in range(16):
                            @pl.when(dvs[k][l] < NOUT)
                            def _edge(l=l):
                                pltpu.sync_copy(table_hbm.at[sv[l]], rowbuf)
                                d = dvs[k][l]
                                acc[d] = acc[d] + rowbuf[...]
            return __
        lax.fori_loop(0, JSUB, sub_body, None)
        return _
    lax.fori_loop(lo, hi, chunk_body, None)

    pltpu.sync_copy(acc, out_hbm.at[wid])


def _agg68(src_flat, dst_flat, table):
    mesh = plsc.VectorSubcoreMesh(core_axis_name="c", subcore_axis_name="s")
    k = pl.kernel(
        _agg68_body,
        out_type=jax.ShapeDtypeStruct((NTILES, OPAD, F), jnp.float32),
        mesh=mesh,
        compiler_params=pltpu.CompilerParams(use_tc_tiling_on_sc=False,
                                             needs_layout_passes=False),
        scratch_types=[
            pltpu.VMEM((CHUNK,), jnp.int32),
            pltpu.VMEM((CHUNK,), jnp.int32),
            pltpu.VMEM((F,), jnp.float32),
            pltpu.VMEM((OPAD, F), jnp.float32),
        ],
    )
    return k(src_flat, dst_flat, table)


BLK = 4000
GRID = N // BLK


def _prep_body(x_ref, w_ref, b_ref, o_ref):
    i = pl.program_id(0)
    rows = (jnp.float32(i * BLK)
            + lax.broadcasted_iota(jnp.int32, (BLK, 1), 0).astype(jnp.float32))
    vect = jnp.tanh(rows * w_ref[...] + b_ref[...])
    o_ref[...] = jnp.concatenate(
        [x_ref[...], vect,
         jnp.ones((BLK, 1), jnp.float32),
         jnp.zeros((BLK, F - 9), jnp.float32)], axis=1)


def _prep(x, pos_W, pos_b):
    return pl.pallas_call(
        _prep_body,
        grid=(GRID,),
        in_specs=[
            pl.BlockSpec((BLK, 3), lambda i: (i, 0)),
            pl.BlockSpec((1, 5), lambda i: (0, 0)),
            pl.BlockSpec((1, 5), lambda i: (0, 0)),
        ],
        out_specs=pl.BlockSpec((BLK, F), lambda i: (i, 0)),
        out_shape=jax.ShapeDtypeStruct((N, F), jnp.float32),
    )(x, pos_W.reshape(1, 5), pos_b.reshape(1, 5))


def _dense1_body(p0, p1, h0, wl, bl, wr, h1_o, rcn_o):
    s8 = p0[:, :8] + p1[:, :8]
    cnt = p0[:, 8:9] + p1[:, 8:9]
    rcn = 1.0 / jnp.maximum(cnt, 1.0)
    h1_o[...] = (jnp.dot(s8 * rcn, wl[...], preferred_element_type=jnp.float32)
                 + bl[...]
                 + jnp.dot(h0[:, :8], wr[...],
                           preferred_element_type=jnp.float32))
    rcn_o[...] = rcn


def _dense1(p0, p1, h0ext, Wl, bl, Wr):
    return pl.pallas_call(
        _dense1_body,
        grid=(GRID,),
        in_specs=[
            pl.BlockSpec((BLK, F), lambda i: (i, 0)),
            pl.BlockSpec((BLK, F), lambda i: (i, 0)),
            pl.BlockSpec((BLK, F), lambda i: (i, 0)),
            pl.BlockSpec((8, F), lambda i: (0, 0)),
            pl.BlockSpec((1, F), lambda i: (0, 0)),
            pl.BlockSpec((8, F), lambda i: (0, 0)),
        ],
        out_specs=[
            pl.BlockSpec((BLK, F), lambda i: (i, 0)),
            pl.BlockSpec((BLK, 1), lambda i: (i, 0)),
        ],
        out_shape=[
            jax.ShapeDtypeStruct((N, F), jnp.float32),
            jax.ShapeDtypeStruct((N, 1), jnp.float32),
        ],
    )(p0, p1, h0ext, Wl.T, bl.reshape(1, F), Wr.T)


def _dense2_body(p0, p1, rcn, h, wl, bl, wr, o_ref):
    agg = (p0[...] + p1[...]) * rcn[...]
    o_ref[...] = (jnp.dot(agg, wl[...], preferred_element_type=jnp.float32)
                  + bl[...]
                  + jnp.dot(h[...], wr[...],
                            preferred_element_type=jnp.float32))


def _dense2(p0, p1, rcn, h, Wl, bl, Wr):
    return pl.pallas_call(
        _dense2_body,
        grid=(GRID,),
        in_specs=[
            pl.BlockSpec((BLK, F), lambda i: (i, 0)),
            pl.BlockSpec((BLK, F), lambda i: (i, 0)),
            pl.BlockSpec((BLK, 1), lambda i: (i, 0)),
            pl.BlockSpec((BLK, F), lambda i: (i, 0)),
            pl.BlockSpec((F, F), lambda i: (0, 0)),
            pl.BlockSpec((1, F), lambda i: (0, 0)),
            pl.BlockSpec((F, F), lambda i: (0, 0)),
        ],
        out_specs=pl.BlockSpec((BLK, F), lambda i: (i, 0)),
        out_shape=jax.ShapeDtypeStruct((N, F), jnp.float32),
    )(p0, p1, rcn, h, Wl.T, bl.reshape(1, F), Wr.T)


def _dense3_body(p, rcn, h, wl, bl, wr, o_ref):
    agg = jnp.sum(p[...], axis=0) * rcn[...]
    o_ref[...] = (jnp.dot(agg, wl[...], preferred_element_type=jnp.float32)
                  + bl[...]
                  + jnp.dot(h[...], wr[...],
                            preferred_element_type=jnp.float32))


def _dense3(p, rcn, h, Wl, bl, Wr):
    return pl.pallas_call(
        _dense3_body,
        out_shape=jax.ShapeDtypeStruct((OPAD, 3), jnp.float32),
    )(p, rcn, h, Wl.T, bl.reshape(1, 3), Wr.T)


def kernel(x, edge_index, pos_W, pos_b,
           Wl1, bl1, Wr1, Wl2, bl2, Wr2, Wl3, bl3, Wr3):
    src = edge_index[0].reshape(NCH, JSUB, SUB)
    dst = edge_index[1].reshape(NCH, JSUB, SUB)

    h0ext = _prep(x, pos_W, pos_b)
    p = _agg(src, dst, h0ext)
    h1, rcn = _dense1(p[0], p[1], h0ext, Wl1, bl1, Wr1)
    p2 = _agg(src, dst, h1)
    h2 = _dense2(p2[0], p2[1], rcn, h1, Wl2, bl2, Wr2)
    p3 = _agg68(edge_index[0], edge_index[1], h2)
    out = _dense3(p3, rcn[:OPAD], h2[:OPAD], Wl3, bl3, Wr3)
    return out[:NOUT]


# speedup vs baseline: 34.5481x; 1.0397x over previous
import functools

import jax
import jax.numpy as jnp
from jax import lax
from jax.experimental import pallas as pl
from jax.experimental.pallas import tpu as pltpu
from jax.experimental.pallas import tpu_sc as plsc

N = 100000
E = 3200000
F = 16
SUB = 128
JSUB = 8
CHUNK = SUB * JSUB
NCH = E // CHUNK
NTILES = 32
RPT = N // 16
ZROWS = 625


NOUT = 68
MP = 102400
MPT = MP // 16


def _agg_body(src_hbm, dst_hbm, table_hbm, out_hbm, mask_hbm,
              idx_s, idx_d, rows0, rows1, zbuf, zmrow, idxsc, valg,
              sem0, sem1, acc, maskp):
    c = lax.axis_index("c")
    s = lax.axis_index("s")
    wid = s * 2 + c

    def zfill(i, _):
        zbuf[i] = jnp.zeros((F,), jnp.float32)
        return _
    lax.fori_loop(0, ZROWS, zfill, None)

    def zmfill(g, _):
        zmrow[pl.ds(g * 16, 16)] = jnp.zeros((16,), jnp.float32)
        return _
    lax.fori_loop(0, MPT // 16, zmfill, None)
    base = s * RPT
    for b in range(RPT // ZROWS):
        pltpu.sync_copy(zbuf, acc.at[pl.ds(base + b * ZROWS, ZROWS)])
    pltpu.sync_copy(zmrow, maskp.at[pl.ds(s * MPT, MPT)])
    plsc.subcore_barrier()

    lo = (wid * NCH) // NTILES
    hi = ((wid + 1) * NCH) // NTILES

    def chunk_body(chunk, _):
        pltpu.sync_copy(src_hbm.at[chunk], idx_s)
        pltpu.sync_copy(dst_hbm.at[chunk], idx_d)
        bufs = (rows0, rows1)
        sems = (sem0, sem1)
        descs = [None] * JSUB
        descs[0] = pltpu.async_copy(table_hbm.at[idx_s.at[0]], bufs[0], sems[0])
        for j in range(JSUB):
            dvs = [idx_d[j, pl.ds(k * 16, 16)] for k in range(8)]
            mins = functools.reduce(jnp.minimum, dvs)

            @pl.when(plsc.all_reduce_population_count(mins < NOUT)[0] > 0)
            def _mhit(j=j, dvs=dvs):
                for k in range(8):
                    @pl.when(plsc.all_reduce_population_count(
                        dvs[k] < NOUT)[0] > 0)
                    def _mgrp(j=j, k=k, dvs=dvs):
                        idxsc[...] = idx_s[j, pl.ds(k * 16, 16)]
                        valg[...] = jnp.where(dvs[k] < NOUT, 1.0, 0.0)
                        pltpu.sync_copy(valg, maskp.at[idxsc], add=True)
        for j in range(JSUB):
            if j + 1 < JSUB:
                descs[j + 1] = pltpu.async_copy(
                    table_hbm.at[idx_s.at[j + 1]], bufs[(j + 1) % 2],
                    sems[(j + 1) % 2])
            descs[j].wait()
            pltpu.sync_copy(bufs[j % 2], acc.at[idx_d.at[j]], add=True)
        return _
    lax.fori_loop(lo, hi, chunk_body, None)
    plsc.subcore_barrier()

    pltpu.sync_copy(acc.at[pl.ds(base, RPT)],
                    out_hbm.at[c, pl.ds(base, RPT)])
    pltpu.sync_copy(maskp.at[pl.ds(s * MPT, MPT)],
                    mask_hbm.at[c, pl.ds(s * MPT, MPT)])


def _agg(src, dst, table):
    mesh = plsc.VectorSubcoreMesh(core_axis_name="c", subcore_axis_name="s")
    k = pl.kernel(
        _agg_body,
        out_type=[jax.ShapeDtypeStruct((2, N, F), jnp.float32),
                  jax.ShapeDtypeStruct((2, MP), jnp.float32)],
        mesh=mesh,
        compiler_params=pltpu.CompilerParams(use_tc_tiling_on_sc=False,
                                             needs_layout_passes=False),
        scratch_types=[
            pltpu.VMEM((JSUB, SUB), jnp.int32),
            pltpu.VMEM((JSUB, SUB), jnp.int32),
            pltpu.VMEM((SUB, F), jnp.float32),
            pltpu.VMEM((SUB, F), jnp.float32),
            pltpu.VMEM((ZROWS, F), jnp.float32),
            pltpu.VMEM((MPT,), jnp.float32),
            pltpu.VMEM((16,), jnp.int32),
            pltpu.VMEM((16,), jnp.float32),
            pltpu.SemaphoreType.DMA,
            pltpu.SemaphoreType.DMA,
            pltpu.VMEM_SHARED((N, F), jnp.float32),
            pltpu.VMEM_SHARED((MP,), jnp.float32),
        ],
    )
    return k(src, dst, table)


NT = N + 16
TRASH = N
STCAP = 160
ZR2 = 625


def _aggm_body(src_hbm, dst_hbm, table_hbm, mask_hbm, out_hbm,
               maskv, srcb, dstb, sst, dstst, dfa, dfb1, dfb2,
               rows, zbuf, cntr, semg, acc):
    c = lax.axis_index("c")
    s = lax.axis_index("s")
    wid = s * 2 + c

    def zfill(i, _):
        zbuf[i] = jnp.zeros((F,), jnp.float32)
        return _
    lax.fori_loop(0, ZR2, zfill, None)
    base = s * RPT
    for b in range(RPT // ZR2):
        pltpu.sync_copy(zbuf, acc.at[pl.ds(base + b * ZR2, ZR2)])
    for g in range(STCAP // 16):
        sst[pl.ds(g * 16, 16)] = jnp.zeros((16,), jnp.int32)
        dstst[pl.ds(g * 16, 16)] = jnp.zeros((16,), jnp.int32)
    cntr[0] = 0
    pltpu.sync_copy(mask_hbm, maskv)
    plsc.subcore_barrier()

    lo = (wid * NCH) // NTILES
    hi = ((wid + 1) * NCH) // NTILES

    def chunk_body(chunk, _):
        pltpu.sync_copy(src_hbm.at[pl.ds(chunk * CHUNK, CHUNK)], srcb)
        pltpu.sync_copy(dst_hbm.at[pl.ds(chunk * CHUNK, CHUNK)], dstb)

        def grp_body(g, __):
            dv = dstb[pl.ds(g * 16, 16)]
            wv = plsc.load_gather(maskv, [jnp.right_shift(dv, 5)])
            hm = jnp.bitwise_and(jnp.right_shift(wv, jnp.bitwise_and(dv, 31)),
                                 1) > 0

            @pl.when(plsc.all_reduce_population_count(hm)[0] > 0)
            def _hit():
                sv = srcb[pl.ds(g * 16, 16)]
                cnt0 = cntr[0]
                plsc.store_compressed(sst.at[pl.ds(cnt0, 16)], sv, mask=hm)
                plsc.store_compressed(dstst.at[pl.ds(cnt0, 16)], dv, mask=hm)
                cntr[0] = cnt0 + plsc.all_reduce_population_count(hm)[0]

                @pl.when(cntr[0] >= SUB)
                def _flush():
                    for t in range(SUB // 16):
                        dfa[pl.ds(t * 16, 16)] = dstst[pl.ds(t * 16, 16)]
                    pltpu.async_copy(table_hbm.at[sst.at[pl.ds(0, SUB)]],
                                     rows.at[pl.ds(0, SUB)], semg).wait()
                    pltpu.sync_copy(rows.at[pl.ds(0, SUB)], acc.at[dfa],
                                    add=True)
                    for t in range(2):
                        tv = sst[pl.ds(SUB + t * 16, 16)]
                        sst[pl.ds(t * 16, 16)] = tv
                        dv2 = dstst[pl.ds(SUB + t * 16, 16)]
                        dstst[pl.ds(t * 16, 16)] = dv2
                    cntr[0] = cntr[0] - SUB
            return __
        lax.fori_loop(0, CHUNK // 16, grp_body, None)
        return _
    lax.fori_loop(lo, hi, chunk_body, None)

    cf = cntr[0]
    for g in range(STCAP // 16):
        dvv = dstst[pl.ds(g * 16, 16)]
        lanev = lax.iota(jnp.int32, 16) + g * 16
        dstst[pl.ds(g * 16, 16)] = jnp.where(lanev < cf, dvv, TRASH)

    @pl.when(cf > 0)
    def _final():
        for t in range(SUB // 16):
            dfb1[pl.ds(t * 16, 16)] = dstst[pl.ds(t * 16, 16)]
        for t in range((STCAP - SUB) // 16):
            dfb2[pl.ds(t * 16, 16)] = dstst[pl.ds(SUB + t * 16, 16)]
        pltpu.async_copy(table_hbm.at[sst.at[pl.ds(0, SUB)]],
                         rows.at[pl.ds(0, SUB)], semg).wait()
        pltpu.async_copy(table_hbm.at[sst.at[pl.ds(SUB, STCAP - SUB)]],
                         rows.at[pl.ds(SUB, STCAP - SUB)], semg).wait()
        pltpu.sync_copy(rows.at[pl.ds(0, SUB)], acc.at[dfb1], add=True)
        pltpu.sync_copy(rows.at[pl.ds(SUB, STCAP - SUB)], acc.at[dfb2],
                        add=True)
    plsc.subcore_barrier()

    pltpu.sync_copy(acc.at[pl.ds(base, RPT)],
                    out_hbm.at[c, pl.ds(base, RPT)])


def _aggm(src_flat, dst_flat, table, maskc):
    mesh = plsc.VectorSubcoreMesh(core_axis_name="c", subcore_axis_name="s")
    k = pl.kernel(
        _aggm_body,
        out_type=jax.ShapeDtypeStruct((2, N, F), jnp.float32),
        mesh=mesh,
        compiler_params=pltpu.CompilerParams(use_tc_tiling_on_sc=False,
                                             needs_layout_passes=False),
        scratch_types=[
            pltpu.VMEM((MP // 32,), jnp.int32),
            pltpu.VMEM((CHUNK,), jnp.int32),
            pltpu.VMEM((CHUNK,), jnp.int32),
            pltpu.VMEM((STCAP,), jnp.int32),
            pltpu.VMEM((STCAP,), jnp.int32),
            pltpu.VMEM((SUB,), jnp.int32),
            pltpu.VMEM((SUB,), jnp.int32),
            pltpu.VMEM((STCAP - SUB,), jnp.int32),
            pltpu.VMEM((STCAP, F), jnp.float32),
            pltpu.VMEM((ZR2, F), jnp.float32),
            pltpu.SMEM((1,), jnp.int32),
            pltpu.SemaphoreType.DMA,
            pltpu.VMEM_SHARED((NT, F), jnp.float32),
        ],
    )
    return k(src_flat, dst_flat, table, maskc)


NOUT = 68
OPAD = 80


def _agg68_body(src_hbm, dst_hbm, table_hbm, out_hbm, srcb, dstb, rowbuf, acc):
    c = lax.axis_index("c")
    s = lax.axis_index("s")
    wid = s * 2 + c

    def zfill(i, _):
        acc[i] = jnp.zeros((F,), jnp.float32)
        return _
    lax.fori_loop(0, OPAD, zfill, None)

    lo = (wid * NCH) // NTILES
    hi = ((wid + 1) * NCH) // NTILES

    def chunk_body(chunk, _):
        pltpu.sync_copy(dst_hbm.at[pl.ds(chunk * CHUNK, CHUNK)], dstb)

        def sub_body(j, __):
            jb = j * SUB
            dvs = [dstb[pl.ds(jb + k * 16, 16)] for k in range(8)]
            mins = functools.reduce(jnp.minimum, dvs)

            @pl.when(plsc.all_reduce_population_count(mins < NOUT)[0] > 0)
            def _hit():
                pltpu.sync_copy(src_hbm.at[pl.ds(chunk * CHUNK, CHUNK)], srcb)
                for k in range(8):
                    @pl.when(plsc.all_reduce_population_count(
                        dvs[k] < NOUT)[0] > 0)
                    def _grp(k=k):
                        sv = srcb[pl.ds(jb + k * 16, 16)]
                        for l in range(16):
                            @pl.when(dvs[k][l] < NOUT)
                            def _edge(l=l):
                                pltpu.sync_copy(table_hbm.at[sv[l]], rowbuf)
                                d = dvs[k][l]
                                acc[d] = acc[d] + rowbuf[...]
            return __
        lax.fori_loop(0, JSUB, sub_body, None)
        return _
    lax.fori_loop(lo, hi, chunk_body, None)

    pltpu.sync_copy(acc, out_hbm.at[wid])


def _agg68(src_flat, dst_flat, table):
    mesh = plsc.VectorSubcoreMesh(core_axis_name="c", subcore_axis_name="s")
    k = pl.kernel(
        _agg68_body,
        out_type=jax.ShapeDtypeStruct((NTILES, OPAD, F), jnp.float32),
        mesh=mesh,
        compiler_params=pltpu.CompilerParams(use_tc_tiling_on_sc=False,
                                             needs_layout_passes=False),
        scratch_types=[
            pltpu.VMEM((CHUNK,), jnp.int32),
            pltpu.VMEM((CHUNK,), jnp.int32),
            pltpu.VMEM((F,), jnp.float32),
            pltpu.VMEM((OPAD, F), jnp.float32),
        ],
    )
    return k(src_flat, dst_flat, table)


BLK = 5000
GRID = N // BLK
MROWS = MP // 128
MBLK = MROWS // GRID


def _prep_body(x_ref, w_ref, b_ref, o_ref):
    i = pl.program_id(0)
    rows = (jnp.float32(i * BLK)
            + lax.broadcasted_iota(jnp.int32, (BLK, 1), 0).astype(jnp.float32))
    vect = jnp.tanh(rows * w_ref[...] + b_ref[...])
    o_ref[...] = jnp.concatenate(
        [x_ref[...], vect,
         jnp.ones((BLK, 1), jnp.float32),
         jnp.zeros((BLK, F - 9), jnp.float32)], axis=1)


def _prep(x, pos_W, pos_b):
    return pl.pallas_call(
        _prep_body,
        grid=(GRID,),
        in_specs=[
            pl.BlockSpec((BLK, 3), lambda i: (i, 0)),
            pl.BlockSpec((1, 5), lambda i: (0, 0)),
            pl.BlockSpec((1, 5), lambda i: (0, 0)),
        ],
        out_specs=pl.BlockSpec((BLK, F), lambda i: (i, 0)),
        out_shape=jax.ShapeDtypeStruct((N, F), jnp.float32),
    )(x, pos_W.reshape(1, 5), pos_b.reshape(1, 5))


def _dense1_body(p0, p1, h0, mps, wl, bl, wr, h1_o, rcn_o, msk_o):
    i = pl.program_id(0)
    s8 = p0[:, :8] + p1[:, :8]
    cnt = p0[:, 8:9] + p1[:, 8:9]
    rcn = 1.0 / jnp.maximum(cnt, 1.0)
    h1_o[...] = (jnp.dot(s8 * rcn, wl[...], preferred_element_type=jnp.float32)
                 + bl[...]
                 + jnp.dot(h0[:, :8], wr[...],
                           preferred_element_type=jnp.float32))
    rcn_o[...] = rcn
    nid = ((i * MBLK + lax.broadcasted_iota(jnp.int32, (MBLK, 128), 0)) * 128
           + lax.broadcasted_iota(jnp.int32, (MBLK, 128), 1))
    mm = mps[0] + mps[1] + (nid < NOUT).astype(jnp.float32)
    bits = jnp.left_shift(
        (mm.reshape(MBLK, 4, 32) > 0.0).astype(jnp.int32),
        lax.broadcasted_iota(jnp.int32, (MBLK, 4, 32), 2))
    msk_o[...] = jnp.sum(bits, axis=2)


def _dense1(p0, p1, h0ext, maskparts, Wl, bl, Wr):
    return pl.pallas_call(
        _dense1_body,
        grid=(GRID,),
        in_specs=[
            pl.BlockSpec((BLK, F), lambda i: (i, 0)),
            pl.BlockSpec((BLK, F), lambda i: (i, 0)),
            pl.BlockSpec((BLK, F), lambda i: (i, 0)),
            pl.BlockSpec((2, MBLK, 128), lambda i: (0, i, 0)),
            pl.BlockSpec((8, F), lambda i: (0, 0)),
            pl.BlockSpec((1, F), lambda i: (0, 0)),
            pl.BlockSpec((8, F), lambda i: (0, 0)),
        ],
        out_specs=[
            pl.BlockSpec((BLK, F), lambda i: (i, 0)),
            pl.BlockSpec((BLK, 1), lambda i: (i, 0)),
            pl.BlockSpec((MBLK, 4), lambda i: (i, 0)),
        ],
        out_shape=[
            jax.ShapeDtypeStruct((N, F), jnp.float32),
            jax.ShapeDtypeStruct((N, 1), jnp.float32),
            jax.ShapeDtypeStruct((MROWS, 4), jnp.int32),
        ],
    )(p0, p1, h0ext, maskparts.reshape(2, MROWS, 128),
      Wl.T, bl.reshape(1, F), Wr.T)


def _dense2_body(p0, p1, rcn, h, wl, bl, wr, o_ref):
    agg = (p0[...] + p1[...]) * rcn[...]
    o_ref[...] = (jnp.dot(agg, wl[...], preferred_element_type=jnp.float32)
                  + bl[...]
                  + jnp.dot(h[...], wr[...],
                            preferred_element_type=jnp.float32))


def _dense2(p0, p1, rcn, h, Wl, bl, Wr):
    return pl.pallas_call(
        _dense2_body,
        grid=(GRID,),
        in_specs=[
            pl.BlockSpec((BLK, F), lambda i: (i, 0)),
            pl.BlockSpec((BLK, F), lambda i: (i, 0)),
            pl.BlockSpec((BLK, 1), lambda i: (i, 0)),
            pl.BlockSpec((BLK, F), lambda i: (i, 0)),
            pl.BlockSpec((F, F), lambda i: (0, 0)),
            pl.BlockSpec((1, F), lambda i: (0, 0)),
            pl.BlockSpec((F, F), lambda i: (0, 0)),
        ],
        out_specs=pl.BlockSpec((BLK, F), lambda i: (i, 0)),
        out_shape=jax.ShapeDtypeStruct((N, F), jnp.float32),
    )(p0, p1, rcn, h, Wl.T, bl.reshape(1, F), Wr.T)


def _dense3_body(p, rcn, h, wl, bl, wr, o_ref):
    agg = jnp.sum(p[...], axis=0) * rcn[...]
    o_ref[...] = (jnp.dot(agg, wl[...], preferred_element_type=jnp.float32)
                  + bl[...]
                  + jnp.dot(h[...], wr[...],
                            preferred_element_type=jnp.float32))


def _dense3(p, rcn, h, Wl, bl, Wr):
    return pl.pallas_call(
        _dense3_body,
        out_shape=jax.ShapeDtypeStruct((OPAD, 3), jnp.float32),
    )(p, rcn, h, Wl.T, bl.reshape(1, 3), Wr.T)


def kernel(x, edge_index, pos_W, pos_b,
           Wl1, bl1, Wr1, Wl2, bl2, Wr2, Wl3, bl3, Wr3):
    src = edge_index[0].reshape(NCH, JSUB, SUB)
    dst = edge_index[1].reshape(NCH, JSUB, SUB)

    h0ext = _prep(x, pos_W, pos_b)
    p, maskparts = _agg(src, dst, h0ext)
    h1, rcn, maskc = _dense1(p[0], p[1], h0ext, maskparts, Wl1, bl1, Wr1)
    p2 = _aggm(edge_index[0], edge_index[1], h1, maskc.reshape(MP // 32))
    h2 = _dense2(p2[0], p2[1], rcn, h1, Wl2, bl2, Wr2)
    p3 = _agg68(edge_index[0], edge_index[1], h2)
    out = _dense3(p3, rcn[:OPAD], h2[:OPAD], Wl3, bl3, Wr3)
    return out[:NOUT]


# speedup vs baseline: 41.5164x; 1.2017x over previous
import functools

import jax
import jax.numpy as jnp
from jax import lax
from jax.experimental import pallas as pl
from jax.experimental.pallas import tpu as pltpu
from jax.experimental.pallas import tpu_sc as plsc

N = 100000
E = 3200000
F = 16
SUB = 128
JSUB = 8
CHUNK = SUB * JSUB
NCH = E // CHUNK
NTILES = 32
RPT = N // 16
ZROWS = 625


NOUT = 68
MP = 102400
MPT = MP // 16


def _agg_body(src_hbm, dst_hbm, table_hbm, zeros_hbm, zmask_hbm,
              out_hbm, mask_hbm,
              idx_s, idx_d, rows, idxsc, valg,
              semi, semg, sems, acc, maskp):
    c = lax.axis_index("c")
    s = lax.axis_index("s")
    wid = s * 2 + c

    base = s * RPT
    pltpu.sync_copy(zeros_hbm.at[pl.ds(base, RPT)], acc.at[pl.ds(base, RPT)])
    pltpu.sync_copy(zmask_hbm.at[pl.ds(s * MPT, MPT)],
                    maskp.at[pl.ds(s * MPT, MPT)])
    plsc.subcore_barrier()

    lo = (wid * NCH) // NTILES
    hi = ((wid + 1) * NCH) // NTILES

    def fire_idx(chunk, b):
        pltpu.async_copy(src_hbm.at[chunk], idx_s.at[b], semi.at[b])
        pltpu.async_copy(dst_hbm.at[chunk], idx_d.at[b], semi.at[b])

    fire_idx(lo, 0)

    def chunk_body(chunk, _):
        b = lax.rem(chunk - lo, 2)
        pltpu.make_async_copy(src_hbm.at[chunk], idx_s.at[b],
                              semi.at[b]).wait()
        pltpu.make_async_copy(dst_hbm.at[chunk], idx_d.at[b],
                              semi.at[b]).wait()

        @pl.when(chunk + 1 < hi)
        def _pref():
            fire_idx(chunk + 1, 1 - b)

        gd = [pltpu.async_copy(table_hbm.at[idx_s.at[b].at[j]],
                               rows.at[j], semg.at[j])
              for j in range(JSUB)]

        for j in range(JSUB):
            dvs = [idx_d[b, j, pl.ds(k * 16, 16)] for k in range(8)]
            mins = functools.reduce(jnp.minimum, dvs)

            @pl.when(plsc.all_reduce_population_count(mins < NOUT)[0] > 0)
            def _mhit(j=j, dvs=dvs):
                for k in range(8):
                    @pl.when(plsc.all_reduce_population_count(
                        dvs[k] < NOUT)[0] > 0)
                    def _mgrp(j=j, k=k, dvs=dvs):
                        idxsc[...] = idx_s[b, j, pl.ds(k * 16, 16)]
                        valg[...] = jnp.where(dvs[k] < NOUT, 1.0, 0.0)
                        pltpu.sync_copy(valg, maskp.at[idxsc], add=True)

        sd = []
        for j in range(JSUB):
            gd[j].wait()
            sd.append(pltpu.async_copy(rows.at[j], acc.at[idx_d.at[b].at[j]],
                                       sems.at[j], add=True))
        for d in sd:
            d.wait()
        return _
    lax.fori_loop(lo, hi, chunk_body, None)
    plsc.subcore_barrier()

    pltpu.sync_copy(acc.at[pl.ds(base, RPT)],
                    out_hbm.at[c, pl.ds(base, RPT)])
    pltpu.sync_copy(maskp.at[pl.ds(s * MPT, MPT)],
                    mask_hbm.at[c, pl.ds(s * MPT, MPT)])


def _agg(src, dst, table, zeros_nf, zeros_mp):
    mesh = plsc.VectorSubcoreMesh(core_axis_name="c", subcore_axis_name="s")
    k = pl.kernel(
        _agg_body,
        out_type=[jax.ShapeDtypeStruct((2, N, F), jnp.float32),
                  jax.ShapeDtypeStruct((2, MP), jnp.float32)],
        mesh=mesh,
        compiler_params=pltpu.CompilerParams(use_tc_tiling_on_sc=False,
                                             needs_layout_passes=False),
        scratch_types=[
            pltpu.VMEM((2, JSUB, SUB), jnp.int32),
            pltpu.VMEM((2, JSUB, SUB), jnp.int32),
            pltpu.VMEM((JSUB, SUB, F), jnp.float32),
            pltpu.VMEM((16,), jnp.int32),
            pltpu.VMEM((16,), jnp.float32),
            pltpu.SemaphoreType.DMA((2,)),
            pltpu.SemaphoreType.DMA((JSUB,)),
            pltpu.SemaphoreType.DMA((JSUB,)),
            pltpu.VMEM_SHARED((N, F), jnp.float32),
            pltpu.VMEM_SHARED((MP,), jnp.float32),
        ],
    )
    return k(src, dst, table, zeros_nf, zeros_mp)


NT = N + 16
TRASH = N
STCAP = 160
ZR2 = 625


def _aggm_body(src_hbm, dst_hbm, table_hbm, mask_hbm, zeros_hbm, out_hbm,
               maskv, srcb, dstb, sst, dstst, dfa, dfb1, dfb2,
               rows, cntr, semg, acc):
    c = lax.axis_index("c")
    s = lax.axis_index("s")
    wid = s * 2 + c

    base = s * RPT
    pltpu.sync_copy(zeros_hbm.at[pl.ds(base, RPT)], acc.at[pl.ds(base, RPT)])
    for g in range(STCAP // 16):
        sst[pl.ds(g * 16, 16)] = jnp.zeros((16,), jnp.int32)
        dstst[pl.ds(g * 16, 16)] = jnp.zeros((16,), jnp.int32)
    cntr[0] = 0
    pltpu.sync_copy(mask_hbm, maskv)
    plsc.subcore_barrier()

    lo = (wid * NCH) // NTILES
    hi = ((wid + 1) * NCH) // NTILES

    def chunk_body(chunk, _):
        pltpu.sync_copy(src_hbm.at[pl.ds(chunk * CHUNK, CHUNK)], srcb)
        pltpu.sync_copy(dst_hbm.at[pl.ds(chunk * CHUNK, CHUNK)], dstb)

        def grp_body(g, __):
            dv = dstb[pl.ds(g * 16, 16)]
            wv = plsc.load_gather(maskv, [jnp.right_shift(dv, 5)])
            hm = jnp.bitwise_and(jnp.right_shift(wv, jnp.bitwise_and(dv, 31)),
                                 1) > 0

            @pl.when(plsc.all_reduce_population_count(hm)[0] > 0)
            def _hit():
                sv = srcb[pl.ds(g * 16, 16)]
                cnt0 = cntr[0]
                plsc.store_compressed(sst.at[pl.ds(cnt0, 16)], sv, mask=hm)
                plsc.store_compressed(dstst.at[pl.ds(cnt0, 16)], dv, mask=hm)
                cntr[0] = cnt0 + plsc.all_reduce_population_count(hm)[0]

                @pl.when(cntr[0] >= SUB)
                def _flush():
                    for t in range(SUB // 16):
                        dfa[pl.ds(t * 16, 16)] = dstst[pl.ds(t * 16, 16)]
                    pltpu.async_copy(table_hbm.at[sst.at[pl.ds(0, SUB)]],
                                     rows.at[pl.ds(0, SUB)], semg).wait()
                    pltpu.sync_copy(rows.at[pl.ds(0, SUB)], acc.at[dfa],
                                    add=True)
                    for t in range(2):
                        tv = sst[pl.ds(SUB + t * 16, 16)]
                        sst[pl.ds(t * 16, 16)] = tv
                        dv2 = dstst[pl.ds(SUB + t * 16, 16)]
                        dstst[pl.ds(t * 16, 16)] = dv2
                    cntr[0] = cntr[0] - SUB
            return __
        lax.fori_loop(0, CHUNK // 16, grp_body, None)
        return _
    lax.fori_loop(lo, hi, chunk_body, None)

    cf = cntr[0]
    for g in range(STCAP // 16):
        dvv = dstst[pl.ds(g * 16, 16)]
        lanev = lax.iota(jnp.int32, 16) + g * 16
        dstst[pl.ds(g * 16, 16)] = jnp.where(lanev < cf, dvv, TRASH)

    @pl.when(cf > 0)
    def _final():
        for t in range(SUB // 16):
            dfb1[pl.ds(t * 16, 16)] = dstst[pl.ds(t * 16, 16)]
        for t in range((STCAP - SUB) // 16):
            dfb2[pl.ds(t * 16, 16)] = dstst[pl.ds(SUB + t * 16, 16)]
        pltpu.async_copy(table_hbm.at[sst.at[pl.ds(0, SUB)]],
                         rows.at[pl.ds(0, SUB)], semg).wait()
        pltpu.async_copy(table_hbm.at[sst.at[pl.ds(SUB, STCAP - SUB)]],
                         rows.at[pl.ds(SUB, STCAP - SUB)], semg).wait()
        pltpu.sync_copy(rows.at[pl.ds(0, SUB)], acc.at[dfb1], add=True)
        pltpu.sync_copy(rows.at[pl.ds(SUB, STCAP - SUB)], acc.at[dfb2],
                        add=True)
    plsc.subcore_barrier()

    pltpu.sync_copy(acc.at[pl.ds(base, RPT)],
                    out_hbm.at[c, pl.ds(base, RPT)])


def _aggm(src_flat, dst_flat, table, maskc, zeros_nf):
    mesh = plsc.VectorSubcoreMesh(core_axis_name="c", subcore_axis_name="s")
    k = pl.kernel(
        _aggm_body,
        out_type=jax.ShapeDtypeStruct((2, N, F), jnp.float32),
        mesh=mesh,
        compiler_params=pltpu.CompilerParams(use_tc_tiling_on_sc=False,
                                             needs_layout_passes=False),
        scratch_types=[
            pltpu.VMEM((MP // 32,), jnp.int32),
            pltpu.VMEM((CHUNK,), jnp.int32),
            pltpu.VMEM((CHUNK,), jnp.int32),
            pltpu.VMEM((STCAP,), jnp.int32),
            pltpu.VMEM((STCAP,), jnp.int32),
            pltpu.VMEM((SUB,), jnp.int32),
            pltpu.VMEM((SUB,), jnp.int32),
            pltpu.VMEM((STCAP - SUB,), jnp.int32),
            pltpu.VMEM((STCAP, F), jnp.float32),
            pltpu.SMEM((1,), jnp.int32),
            pltpu.SemaphoreType.DMA,
            pltpu.VMEM_SHARED((NT, F), jnp.float32),
        ],
    )
    return k(src_flat, dst_flat, table, maskc, zeros_nf)


NOUT = 68
OPAD = 80


def _agg68_body(src_hbm, dst_hbm, table_hbm, out_hbm, srcb, dstb, rowbuf, acc):
    c = lax.axis_index("c")
    s = lax.axis_index("s")
    wid = s * 2 + c

    def zfill(i, _):
        acc[i] = jnp.zeros((F,), jnp.float32)
        return _
    lax.fori_loop(0, OPAD, zfill, None)

    lo = (wid * NCH) // NTILES
    hi = ((wid + 1) * NCH) // NTILES

    def chunk_body(chunk, _):
        pltpu.sync_copy(dst_hbm.at[pl.ds(chunk * CHUNK, CHUNK)], dstb)

        def sub_body(j, __):
            jb = j * SUB
            dvs = [dstb[pl.ds(jb + k * 16, 16)] for k in range(8)]
            mins = functools.reduce(jnp.minimum, dvs)

            @pl.when(plsc.all_reduce_population_count(mins < NOUT)[0] > 0)
            def _hit():
                pltpu.sync_copy(src_hbm.at[pl.ds(chunk * CHUNK, CHUNK)], srcb)
                for k in range(8):
                    @pl.when(plsc.all_reduce_population_count(
                        dvs[k] < NOUT)[0] > 0)
                    def _grp(k=k):
                        sv = srcb[pl.ds(jb + k * 16, 16)]
                        for l in range(16):
                            @pl.when(dvs[k][l] < NOUT)
                            def _edge(l=l):
                                pltpu.sync_copy(table_hbm.at[sv[l]], rowbuf)
                                d = dvs[k][l]
                                acc[d] = acc[d] + rowbuf[...]
            return __
        lax.fori_loop(0, JSUB, sub_body, None)
        return _
    lax.fori_loop(lo, hi, chunk_body, None)

    pltpu.sync_copy(acc, out_hbm.at[wid])


def _agg68(src_flat, dst_flat, table):
    mesh = plsc.VectorSubcoreMesh(core_axis_name="c", subcore_axis_name="s")
    k = pl.kernel(
        _agg68_body,
        out_type=jax.ShapeDtypeStruct((NTILES, OPAD, F), jnp.float32),
        mesh=mesh,
        compiler_params=pltpu.CompilerParams(use_tc_tiling_on_sc=False,
                                             needs_layout_passes=False),
        scratch_types=[
            pltpu.VMEM((CHUNK,), jnp.int32),
            pltpu.VMEM((CHUNK,), jnp.int32),
            pltpu.VMEM((F,), jnp.float32),
            pltpu.VMEM((OPAD, F), jnp.float32),
        ],
    )
    return k(src_flat, dst_flat, table)


BLK = 5000
GRID = N // BLK
MROWS = MP // 128
MBLK = MROWS // GRID


def _prep_body(x_ref, w_ref, b_ref, o_ref):
    i = pl.program_id(0)
    rows = (jnp.float32(i * BLK)
            + lax.broadcasted_iota(jnp.int32, (BLK, 1), 0).astype(jnp.float32))
    vect = jnp.tanh(rows * w_ref[...] + b_ref[...])
    o_ref[...] = jnp.concatenate(
        [x_ref[...], vect,
         jnp.ones((BLK, 1), jnp.float32),
         jnp.zeros((BLK, F - 9), jnp.float32)], axis=1)


def _prep(x, pos_W, pos_b):
    return pl.pallas_call(
        _prep_body,
        grid=(GRID,),
        in_specs=[
            pl.BlockSpec((BLK, 3), lambda i: (i, 0)),
            pl.BlockSpec((1, 5), lambda i: (0, 0)),
            pl.BlockSpec((1, 5), lambda i: (0, 0)),
        ],
        out_specs=pl.BlockSpec((BLK, F), lambda i: (i, 0)),
        out_shape=jax.ShapeDtypeStruct((N, F), jnp.float32),
    )(x, pos_W.reshape(1, 5), pos_b.reshape(1, 5))


def _dense1_body(p0, p1, h0, mps, wl, bl, wr, h1_o, rcn_o, msk_o):
    i = pl.program_id(0)
    s8 = p0[:, :8] + p1[:, :8]
    cnt = p0[:, 8:9] + p1[:, 8:9]
    rcn = 1.0 / jnp.maximum(cnt, 1.0)
    h1_o[...] = (jnp.dot(s8 * rcn, wl[...], preferred_element_type=jnp.float32)
                 + bl[...]
                 + jnp.dot(h0[:, :8], wr[...],
                           preferred_element_type=jnp.float32))
    rcn_o[...] = rcn
    nid = ((i * MBLK + lax.broadcasted_iota(jnp.int32, (MBLK, 128), 0)) * 128
           + lax.broadcasted_iota(jnp.int32, (MBLK, 128), 1))
    mm = mps[0] + mps[1] + (nid < NOUT).astype(jnp.float32)
    bits = jnp.left_shift(
        (mm.reshape(MBLK, 4, 32) > 0.0).astype(jnp.int32),
        lax.broadcasted_iota(jnp.int32, (MBLK, 4, 32), 2))
    msk_o[...] = jnp.sum(bits, axis=2)


def _dense1(p0, p1, h0ext, maskparts, Wl, bl, Wr):
    return pl.pallas_call(
        _dense1_body,
        grid=(GRID,),
        in_specs=[
            pl.BlockSpec((BLK, F), lambda i: (i, 0)),
            pl.BlockSpec((BLK, F), lambda i: (i, 0)),
            pl.BlockSpec((BLK, F), lambda i: (i, 0)),
            pl.BlockSpec((2, MBLK, 128), lambda i: (0, i, 0)),
            pl.BlockSpec((8, F), lambda i: (0, 0)),
            pl.BlockSpec((1, F), lambda i: (0, 0)),
            pl.BlockSpec((8, F), lambda i: (0, 0)),
        ],
        out_specs=[
            pl.BlockSpec((BLK, F), lambda i: (i, 0)),
            pl.BlockSpec((BLK, 1), lambda i: (i, 0)),
            pl.BlockSpec((MBLK, 4), lambda i: (i, 0)),
        ],
        out_shape=[
            jax.ShapeDtypeStruct((N, F), jnp.float32),
            jax.ShapeDtypeStruct((N, 1), jnp.float32),
            jax.ShapeDtypeStruct((MROWS, 4), jnp.int32),
        ],
    )(p0, p1, h0ext, maskparts.reshape(2, MROWS, 128),
      Wl.T, bl.reshape(1, F), Wr.T)


def _dense2_body(p0, p1, rcn, h, wl, bl, wr, o_ref):
    agg = (p0[...] + p1[...]) * rcn[...]
    o_ref[...] = (jnp.dot(agg, wl[...], preferred_element_type=jnp.float32)
                  + bl[...]
                  + jnp.dot(h[...], wr[...],
                            preferred_element_type=jnp.float32))


def _dense2(p0, p1, rcn, h, Wl, bl, Wr):
    return pl.pallas_call(
        _dense2_body,
        grid=(GRID,),
        in_specs=[
            pl.BlockSpec((BLK, F), lambda i: (i, 0)),
            pl.BlockSpec((BLK, F), lambda i: (i, 0)),
            pl.BlockSpec((BLK, 1), lambda i: (i, 0)),
            pl.BlockSpec((BLK, F), lambda i: (i, 0)),
            pl.BlockSpec((F, F), lambda i: (0, 0)),
            pl.BlockSpec((1, F), lambda i: (0, 0)),
            pl.BlockSpec((F, F), lambda i: (0, 0)),
        ],
        out_specs=pl.BlockSpec((BLK, F), lambda i: (i, 0)),
        out_shape=jax.ShapeDtypeStruct((N, F), jnp.float32),
    )(p0, p1, rcn, h, Wl.T, bl.reshape(1, F), Wr.T)


def _dense3_body(p, rcn, h, wl, bl, wr, o_ref):
    agg = jnp.sum(p[...], axis=0) * rcn[...]
    o_ref[...] = (jnp.dot(agg, wl[...], preferred_element_type=jnp.float32)
                  + bl[...]
                  + jnp.dot(h[...], wr[...],
                            preferred_element_type=jnp.float32))


def _dense3(p, rcn, h, Wl, bl, Wr):
    return pl.pallas_call(
        _dense3_body,
        out_shape=jax.ShapeDtypeStruct((OPAD, 3), jnp.float32),
    )(p, rcn, h, Wl.T, bl.reshape(1, 3), Wr.T)


def kernel(x, edge_index, pos_W, pos_b,
           Wl1, bl1, Wr1, Wl2, bl2, Wr2, Wl3, bl3, Wr3):
    src = edge_index[0].reshape(NCH, JSUB, SUB)
    dst = edge_index[1].reshape(NCH, JSUB, SUB)

    zeros_nf = jnp.zeros((N, F), jnp.float32)
    zeros_mp = jnp.zeros((MP,), jnp.float32)
    h0ext = _prep(x, pos_W, pos_b)
    p, maskparts = _agg(src, dst, h0ext, zeros_nf, zeros_mp)
    h1, rcn, maskc = _dense1(p[0], p[1], h0ext, maskparts, Wl1, bl1, Wr1)
    p2 = _aggm(edge_index[0], edge_index[1], h1, maskc.reshape(MP // 32),
               zeros_nf)
    h2 = _dense2(p2[0], p2[1], rcn, h1, Wl2, bl2, Wr2)
    p3 = _agg68(edge_index[0], edge_index[1], h2)
    out = _dense3(p3, rcn[:OPAD], h2[:OPAD], Wl3, bl3, Wr3)
    return out[:NOUT]


# speedup vs baseline: 61.1118x; 1.4720x over previous
import functools

import jax
import jax.numpy as jnp
from jax import lax
from jax.experimental import pallas as pl
from jax.experimental.pallas import tpu as pltpu
from jax.experimental.pallas import tpu_sc as plsc

N = 100000
E = 3200000
F = 16
SUB = 128
JSUB = 10
CHUNK = SUB * JSUB
NCH = E // CHUNK
NTILES = 32
RPT = N // 16
ZROWS = 625


def _agg_body(src_hbm, dst_hbm, table_hbm, zeros_hbm,
              out_hbm,
              idx_s, idx_d, rows,
              semi, semg, sems, acc):
    c = lax.axis_index("c")
    s = lax.axis_index("s")
    wid = s * 2 + c

    base = s * RPT
    pltpu.sync_copy(zeros_hbm.at[pl.ds(base, RPT)], acc.at[pl.ds(base, RPT)])
    plsc.subcore_barrier()

    lo = (wid * NCH) // NTILES
    hi = ((wid + 1) * NCH) // NTILES

    def fire_idx(chunk, b):
        pltpu.async_copy(src_hbm.at[chunk], idx_s.at[b], semi.at[b])
        pltpu.async_copy(dst_hbm.at[chunk], idx_d.at[b], semi.at[b])

    fire_idx(lo, 0)

    def chunk_body(chunk, _):
        b = lax.rem(chunk - lo, 2)
        pltpu.make_async_copy(src_hbm.at[chunk], idx_s.at[b],
                              semi.at[b]).wait()
        pltpu.make_async_copy(dst_hbm.at[chunk], idx_d.at[b],
                              semi.at[b]).wait()

        @pl.when(chunk + 1 < hi)
        def _pref():
            fire_idx(chunk + 1, 1 - b)

        gd = [pltpu.async_copy(table_hbm.at[idx_s.at[b].at[j]],
                               rows.at[j], semg.at[j])
              for j in range(JSUB)]

        sd = []
        for j in range(JSUB):
            gd[j].wait()
            sd.append(pltpu.async_copy(rows.at[j], acc.at[idx_d.at[b].at[j]],
                                       sems.at[j], add=True))
        for d in sd:
            d.wait()
        return _
    lax.fori_loop(lo, hi, chunk_body, None)
    plsc.subcore_barrier()

    pltpu.sync_copy(acc.at[pl.ds(base, RPT)],
                    out_hbm.at[c, pl.ds(base, RPT)])


def _agg(src, dst, table, zeros_nf):
    mesh = plsc.VectorSubcoreMesh(core_axis_name="c", subcore_axis_name="s")
    k = pl.kernel(
        _agg_body,
        out_type=jax.ShapeDtypeStruct((2, N, F), jnp.float32),
        mesh=mesh,
        compiler_params=pltpu.CompilerParams(use_tc_tiling_on_sc=False,
                                             needs_layout_passes=False),
        scratch_types=[
            pltpu.VMEM((2, JSUB, SUB), jnp.int32),
            pltpu.VMEM((2, JSUB, SUB), jnp.int32),
            pltpu.VMEM((JSUB, SUB, F), jnp.float32),
            pltpu.SemaphoreType.DMA((2,)),
            pltpu.SemaphoreType.DMA((JSUB,)),
            pltpu.SemaphoreType.DMA((JSUB,)),
            pltpu.VMEM_SHARED((N, F), jnp.float32),
        ],
    )
    return k(src, dst, table, zeros_nf)


NOUT = 68
OPAD = 80


def _agg68_body(src_hbm, dst_hbm, table_hbm, out_hbm,
                srcb, dstb, rowbuf, acc, semi):
    c = lax.axis_index("c")
    s = lax.axis_index("s")
    wid = s * 2 + c

    def zfill(i, _):
        acc[i] = jnp.zeros((F,), jnp.float32)
        return _
    lax.fori_loop(0, OPAD, zfill, None)

    lo = (wid * NCH) // NTILES
    hi = ((wid + 1) * NCH) // NTILES

    def fire_idx(chunk, b):
        pltpu.async_copy(src_hbm.at[chunk], srcb.at[b], semi.at[b])
        pltpu.async_copy(dst_hbm.at[chunk], dstb.at[b], semi.at[b])

    fire_idx(lo, 0)

    def chunk_body(chunk, _):
        b = lax.rem(chunk - lo, 2)
        pltpu.make_async_copy(src_hbm.at[chunk], srcb.at[b],
                              semi.at[b]).wait()
        pltpu.make_async_copy(dst_hbm.at[chunk], dstb.at[b],
                              semi.at[b]).wait()

        @pl.when(chunk + 1 < hi)
        def _pref():
            fire_idx(chunk + 1, 1 - b)

        def sub_body(j, __):
            dvs = [dstb[b, j, pl.ds(k * 16, 16)] for k in range(8)]
            mins = functools.reduce(jnp.minimum, dvs)

            @pl.when(plsc.all_reduce_population_count(mins < NOUT)[0] > 0)
            def _hit():
                for k in range(8):
                    @pl.when(plsc.all_reduce_population_count(
                        dvs[k] < NOUT)[0] > 0)
                    def _grp(k=k):
                        sv = srcb[b, j, pl.ds(k * 16, 16)]
                        for l in range(16):
                            @pl.when(dvs[k][l] < NOUT)
                            def _edge(l=l):
                                pltpu.sync_copy(table_hbm.at[sv[l]], rowbuf)
                                d = dvs[k][l]
                                acc[d] = acc[d] + rowbuf[...]
            return __
        lax.fori_loop(0, JSUB, sub_body, None)
        return _
    lax.fori_loop(lo, hi, chunk_body, None)

    pltpu.sync_copy(acc, out_hbm.at[wid])


def _agg68(src, dst, table):
    mesh = plsc.VectorSubcoreMesh(core_axis_name="c", subcore_axis_name="s")
    k = pl.kernel(
        _agg68_body,
        out_type=jax.ShapeDtypeStruct((NTILES, OPAD, F), jnp.float32),
        mesh=mesh,
        compiler_params=pltpu.CompilerParams(use_tc_tiling_on_sc=False,
                                             needs_layout_passes=False),
        scratch_types=[
            pltpu.VMEM((2, JSUB, SUB), jnp.int32),
            pltpu.VMEM((2, JSUB, SUB), jnp.int32),
            pltpu.VMEM((F,), jnp.float32),
            pltpu.VMEM((OPAD, F), jnp.float32),
            pltpu.SemaphoreType.DMA((2,)),
        ],
    )
    return k(src, dst, table)


BLK = 5000
GRID = N // BLK


def _prep_body(x_ref, w_ref, b_ref, o_ref):
    i = pl.program_id(0)
    rows = (jnp.float32(i * BLK)
            + lax.broadcasted_iota(jnp.int32, (BLK, 1), 0).astype(jnp.float32))
    vect = jnp.tanh(rows * w_ref[...] + b_ref[...])
    o_ref[...] = jnp.concatenate(
        [x_ref[...], vect,
         jnp.ones((BLK, 1), jnp.float32),
         jnp.zeros((BLK, F - 9), jnp.float32)], axis=1)


def _prep(x, pos_W, pos_b):
    return pl.pallas_call(
        _prep_body,
        grid=(GRID,),
        in_specs=[
            pl.BlockSpec((BLK, 3), lambda i: (i, 0)),
            pl.BlockSpec((1, 5), lambda i: (0, 0)),
            pl.BlockSpec((1, 5), lambda i: (0, 0)),
        ],
        out_specs=pl.BlockSpec((BLK, F), lambda i: (i, 0)),
        out_shape=jax.ShapeDtypeStruct((N, F), jnp.float32),
    )(x, pos_W.reshape(1, 5), pos_b.reshape(1, 5))


def _dense1_body(p, h0, wl, bl, wr, h1_o, rcn_o):
    s8 = p[0, :, :8] + p[1, :, :8]
    cnt = p[0, :, 8:9] + p[1, :, 8:9]
    rcn = 1.0 / jnp.maximum(cnt, 1.0)
    h1_o[...] = (jnp.dot(s8 * rcn, wl[...], preferred_element_type=jnp.float32)
                 + bl[...]
                 + jnp.dot(h0[:, :8], wr[...],
                           preferred_element_type=jnp.float32))
    rcn_o[...] = rcn


def _dense1(p, h0ext, Wl, bl, Wr):
    return pl.pallas_call(
        _dense1_body,
        grid=(GRID,),
        in_specs=[
            pl.BlockSpec((2, BLK, F), lambda i: (0, i, 0)),
            pl.BlockSpec((BLK, F), lambda i: (i, 0)),
            pl.BlockSpec((8, F), lambda i: (0, 0)),
            pl.BlockSpec((1, F), lambda i: (0, 0)),
            pl.BlockSpec((8, F), lambda i: (0, 0)),
        ],
        out_specs=[
            pl.BlockSpec((BLK, F), lambda i: (i, 0)),
            pl.BlockSpec((BLK, 1), lambda i: (i, 0)),
        ],
        out_shape=[
            jax.ShapeDtypeStruct((N, F), jnp.float32),
            jax.ShapeDtypeStruct((N, 1), jnp.float32),
        ],
    )(p, h0ext, Wl.T, bl.reshape(1, F), Wr.T)


def _dense2_body(p, rcn, h, wl, bl, wr, o_ref):
    agg = (p[0] + p[1]) * rcn[...]
    o_ref[...] = (jnp.dot(agg, wl[...], preferred_element_type=jnp.float32)
                  + bl[...]
                  + jnp.dot(h[...], wr[...],
                            preferred_element_type=jnp.float32))


def _dense2(p, rcn, h, Wl, bl, Wr):
    return pl.pallas_call(
        _dense2_body,
        grid=(GRID,),
        in_specs=[
            pl.BlockSpec((2, BLK, F), lambda i: (0, i, 0)),
            pl.BlockSpec((BLK, 1), lambda i: (i, 0)),
            pl.BlockSpec((BLK, F), lambda i: (i, 0)),
            pl.BlockSpec((F, F), lambda i: (0, 0)),
            pl.BlockSpec((1, F), lambda i: (0, 0)),
            pl.BlockSpec((F, F), lambda i: (0, 0)),
        ],
        out_specs=pl.BlockSpec((BLK, F), lambda i: (i, 0)),
        out_shape=jax.ShapeDtypeStruct((N, F), jnp.float32),
    )(p, rcn, h, Wl.T, bl.reshape(1, F), Wr.T)


def _dense3_body(p, rcn, h, wl, bl, wr, o_ref):
    agg = jnp.sum(p[...], axis=0) * rcn[...]
    o_ref[...] = (jnp.dot(agg, wl[...], preferred_element_type=jnp.float32)
                  + bl[...]
                  + jnp.dot(h[...], wr[...],
                            preferred_element_type=jnp.float32))


def _dense3(p, rcn, h, Wl, bl, Wr):
    return pl.pallas_call(
        _dense3_body,
        out_shape=jax.ShapeDtypeStruct((OPAD, 3), jnp.float32),
    )(p, rcn, h, Wl.T, bl.reshape(1, 3), Wr.T)


def kernel(x, edge_index, pos_W, pos_b,
           Wl1, bl1, Wr1, Wl2, bl2, Wr2, Wl3, bl3, Wr3):
    src = edge_index[0].reshape(NCH, JSUB, SUB)
    dst = edge_index[1].reshape(NCH, JSUB, SUB)
    zeros_nf = jnp.zeros((N, F), jnp.float32)

    h0ext = _prep(x, pos_W, pos_b)
    p = _agg(src, dst, h0ext, zeros_nf)
    h1, rcn = _dense1(p, h0ext, Wl1, bl1, Wr1)
    p2 = _agg(src, dst, h1, zeros_nf)
    h2 = _dense2(p2, rcn, h1, Wl2, bl2, Wr2)
    p3 = _agg68(src, dst, h2)
    out = _dense3(p3, rcn[:OPAD], h2[:OPAD], Wl3, bl3, Wr3)
    return out[:NOUT]
